# Initial kernel scaffold; baseline (speedup 1.0000x reference)
import jax
import jax.numpy as jnp
from jax.experimental import pallas as pl


def kernel(x_cont, x_cat, ln_gamma, ln_beta, W_cont, b_cont, tables, W_cat, b_cat, W_fin, b_fin):
    raise NotImplementedError("write your pallas kernel here")


# speedup vs baseline: 6.1059x; 6.1059x over previous
import functools

import jax
import jax.numpy as jnp
from jax import lax
from jax.experimental import pallas as pl
from jax.experimental.pallas import tpu as pltpu
from jax.experimental.pallas import tpu_sc as plsc

N_CAT = 26
VOCAB = 100000
EMB = 50
D_HALF = 64
D_MODEL = 128
B, S = 4096, 50
T = B * S

_P1_BLK = 25000


def _proj_body(tab_ref, w_ref, out_ref):
    out_ref[...] = jnp.dot(tab_ref[0], w_ref[0],
                           preferred_element_type=jnp.float32)


def _project_tables(tables, w_cat3):
    nblk = VOCAB // _P1_BLK
    return pl.pallas_call(
        _proj_body,
        grid=(N_CAT, nblk),
        in_specs=[
            pl.BlockSpec((1, _P1_BLK, EMB), lambda i, j: (i, j, 0)),
            pl.BlockSpec((1, EMB, D_HALF), lambda i, j: (i, 0, 0)),
        ],
        out_specs=pl.BlockSpec((_P1_BLK, D_HALF),
                               lambda i, j, _n=nblk: (i * _n + j, 0)),
        out_shape=jax.ShapeDtypeStruct((N_CAT * VOCAB, D_HALF), jnp.float32),
    )(tables, w_cat3)


_NC, _NS, _L = 2, 16, 16
_NW = _NC * _NS
_TPW = T // _NW
_CT = 64
_GPC = _CT * N_CAT
_NGB = _GPC // 128
_NCH = _TPW // _CT


def _gather_sum_body(idx_hbm, proj_hbm, out_hbm, idx_v, rows_v, acc_v, sem):
    wid = lax.axis_index("s") * _NC + lax.axis_index("c")
    base = wid * _TPW

    def chunk(ci, carry):
        tok0 = base + ci * _CT
        pltpu.sync_copy(idx_hbm.at[pl.ds(tok0 * N_CAT, _GPC)], idx_v)
        cps = [
            pltpu.async_copy(
                proj_hbm.at[idx_v.at[pl.ds(g * 128, 128)]],
                rows_v.at[pl.ds(g * 128, 128)],
                sem,
            )
            for g in range(_NGB)
        ]
        for cp in cps:
            cp.wait()

        def tok(t, c2):
            r0 = t * N_CAT
            for k in range(D_HALF // _L):
                s = rows_v[r0, pl.ds(k * _L, _L)]
                for j in range(1, N_CAT):
                    s = s + rows_v[r0 + j, pl.ds(k * _L, _L)]
                acc_v[t, pl.ds(k * _L, _L)] = s
            return c2

        lax.fori_loop(0, _CT, tok, 0, unroll=False)
        pltpu.sync_copy(acc_v, out_hbm.at[pl.ds(tok0, _CT)])
        return carry

    lax.fori_loop(0, _NCH, chunk, 0, unroll=False)


def _gather_sum(idx_flat, proj):
    mesh = plsc.VectorSubcoreMesh(core_axis_name="c", subcore_axis_name="s")
    return pl.kernel(
        _gather_sum_body,
        mesh=mesh,
        compiler_params=pltpu.CompilerParams(use_tc_tiling_on_sc=False),
        out_type=jax.ShapeDtypeStruct((T, D_HALF), jnp.float32),
        scratch_types=[
            pltpu.VMEM((_GPC,), jnp.int32),
            pltpu.VMEM((_GPC, D_HALF), jnp.float32),
            pltpu.VMEM((_CT, D_HALF), jnp.float32),
            pltpu.SemaphoreType.DMA,
        ],
    )(idx_flat, proj)


_P3_BT = 4096


def _final_body(x_ref, cs_ref, g_ref, b_ref, wc_ref, bc_ref, bcat_ref,
                wf_ref, bf_ref, out_ref):
    x = x_ref[...]
    mean = jnp.mean(x, axis=1, keepdims=True)
    cen = x - mean
    var = jnp.mean(cen * cen, axis=1, keepdims=True)
    xn = cen * lax.rsqrt(var + 1e-5) * g_ref[...] + b_ref[...]
    ce = jnp.dot(xn, wc_ref[...],
                 preferred_element_type=jnp.float32) + bc_ref[...]
    cat = cs_ref[...] + bcat_ref[...]
    comb = jnp.concatenate([ce, cat], axis=1)
    out_ref[...] = jnp.dot(comb, wf_ref[...],
                           preferred_element_type=jnp.float32) + bf_ref[...]


def _finalize(x2, cat_sum, ln_gamma, ln_beta, W_cont, b_cont, b_cat,
              W_fin, b_fin):
    nblk = T // _P3_BT
    full = lambda i: (0, 0)
    return pl.pallas_call(
        _final_body,
        grid=(nblk,),
        in_specs=[
            pl.BlockSpec((_P3_BT, 13), lambda i: (i, 0)),
            pl.BlockSpec((_P3_BT, D_HALF), lambda i: (i, 0)),
            pl.BlockSpec((1, 13), full),
            pl.BlockSpec((1, 13), full),
            pl.BlockSpec((13, D_HALF), full),
            pl.BlockSpec((1, D_HALF), full),
            pl.BlockSpec((1, D_HALF), full),
            pl.BlockSpec((D_MODEL, D_MODEL), full),
            pl.BlockSpec((1, D_MODEL), full),
        ],
        out_specs=pl.BlockSpec((_P3_BT, D_MODEL), lambda i: (i, 0)),
        out_shape=jax.ShapeDtypeStruct((T, D_MODEL), jnp.float32),
    )(x2, cat_sum, ln_gamma.reshape(1, 13), ln_beta.reshape(1, 13),
      W_cont, b_cont.reshape(1, D_HALF), b_cat.reshape(1, D_HALF),
      W_fin, b_fin.reshape(1, D_MODEL))


def kernel(x_cont, x_cat, ln_gamma, ln_beta, W_cont, b_cont, tables, W_cat,
           b_cat, W_fin, b_fin):
    w_cat3 = W_cat.reshape(N_CAT, EMB, D_HALF)
    proj = _project_tables(tables, w_cat3)

    idx_flat = (x_cat.reshape(T, N_CAT).astype(jnp.int32)
                + (jnp.arange(N_CAT, dtype=jnp.int32) * VOCAB)).reshape(-1)
    cat_sum = _gather_sum(idx_flat, proj)

    x2 = x_cont.reshape(T, 13)
    out = _finalize(x2, cat_sum, ln_gamma, ln_beta, W_cont, b_cont, b_cat,
                    W_fin, b_fin)
    return out.reshape(B, S, D_MODEL)


# speedup vs baseline: 7.0710x; 1.1581x over previous
import functools

import jax
import jax.numpy as jnp
from jax import lax
from jax.experimental import pallas as pl
from jax.experimental.pallas import tpu as pltpu
from jax.experimental.pallas import tpu_sc as plsc

N_CAT = 26
VOCAB = 100000
EMB = 50
D_HALF = 64
D_MODEL = 128
B, S = 4096, 50
T = B * S

_P1_BLK = 10000


def _proj_body(tab_ref, w_ref, out_ref):
    out_ref[...] = jnp.dot(tab_ref[0], w_ref[0],
                           preferred_element_type=jnp.float32)


def _project_tables(tables, w_cat3):
    tab2 = tables.reshape(N_CAT, VOCAB // 2, 2 * EMB)
    w2 = jnp.zeros((N_CAT, 2 * EMB, 2 * D_HALF), jnp.float32)
    w2 = w2.at[:, :EMB, :D_HALF].set(w_cat3).at[:, EMB:, D_HALF:].set(w_cat3)
    nblk = (VOCAB // 2) // _P1_BLK
    proj2 = pl.pallas_call(
        _proj_body,
        grid=(N_CAT, nblk),
        in_specs=[
            pl.BlockSpec((1, _P1_BLK, 2 * EMB), lambda i, j: (i, j, 0)),
            pl.BlockSpec((1, 2 * EMB, 2 * D_HALF), lambda i, j: (i, 0, 0)),
        ],
        out_specs=pl.BlockSpec((_P1_BLK, 2 * D_HALF),
                               lambda i, j, _n=nblk: (i * _n + j, 0)),
        out_shape=jax.ShapeDtypeStruct((N_CAT * VOCAB // 2, 2 * D_HALF),
                                       jnp.float32),
    )(tab2, w2)
    return proj2


_NC, _NS, _L = 2, 16, 16
_NW = _NC * _NS
_TPW = T // _NW
_CT = 64
_GPC = _CT * N_CAT
_NGB = _GPC // 128
_NCH = _TPW // _CT


def _gather_sum_body(idx_hbm, proj_hbm, out_hbm, idx_v, rows_v, acc_v, sem):
    wid = lax.axis_index("s") * _NC + lax.axis_index("c")
    base = wid * _TPW

    def chunk(ci, carry):
        tok0 = base + ci * _CT
        pltpu.sync_copy(idx_hbm.at[pl.ds(tok0 * N_CAT, _GPC)], idx_v)
        cps = [
            pltpu.async_copy(
                proj_hbm.at[idx_v.at[pl.ds(g * 128, 128)]],
                rows_v.at[pl.ds(g * 128, 128)],
                sem,
            )
            for g in range(_NGB)
        ]
        for cp in cps:
            cp.wait()

        def tok(t, c2):
            r0 = t * N_CAT
            for k in range(D_HALF // _L):
                s = rows_v[r0, pl.ds(k * _L, _L)]
                for j in range(1, N_CAT):
                    s = s + rows_v[r0 + j, pl.ds(k * _L, _L)]
                acc_v[t, pl.ds(k * _L, _L)] = s
            return c2

        lax.fori_loop(0, _CT, tok, 0, unroll=False)
        pltpu.sync_copy(acc_v, out_hbm.at[pl.ds(tok0, _CT)])
        return carry

    lax.fori_loop(0, _NCH, chunk, 0, unroll=False)


def _gather_sum(idx_flat, proj):
    mesh = plsc.VectorSubcoreMesh(core_axis_name="c", subcore_axis_name="s")
    return pl.kernel(
        _gather_sum_body,
        mesh=mesh,
        compiler_params=pltpu.CompilerParams(use_tc_tiling_on_sc=False),
        out_type=jax.ShapeDtypeStruct((T, D_HALF), jnp.float32),
        scratch_types=[
            pltpu.VMEM((_GPC,), jnp.int32),
            pltpu.VMEM((_GPC, D_HALF), jnp.float32),
            pltpu.VMEM((_CT, D_HALF), jnp.float32),
            pltpu.SemaphoreType.DMA,
        ],
    )(idx_flat, proj)


_P3_BT = 4096


def _final_body(x_ref, cs_ref, g_ref, b_ref, wc_ref, bc_ref, bcat_ref,
                wf_ref, bf_ref, out_ref):
    x = x_ref[...]
    mean = jnp.mean(x, axis=1, keepdims=True)
    cen = x - mean
    var = jnp.mean(cen * cen, axis=1, keepdims=True)
    xn = cen * lax.rsqrt(var + 1e-5) * g_ref[...] + b_ref[...]
    ce = jnp.dot(xn, wc_ref[...],
                 preferred_element_type=jnp.float32) + bc_ref[...]
    cat = cs_ref[...] + bcat_ref[...]
    comb = jnp.concatenate([ce, cat], axis=1)
    out_ref[...] = jnp.dot(comb, wf_ref[...],
                           preferred_element_type=jnp.float32) + bf_ref[...]


def _finalize(x2, cat_sum, ln_gamma, ln_beta, W_cont, b_cont, b_cat,
              W_fin, b_fin):
    nblk = T // _P3_BT
    full = lambda i: (0, 0)
    return pl.pallas_call(
        _final_body,
        grid=(nblk,),
        in_specs=[
            pl.BlockSpec((_P3_BT, 13), lambda i: (i, 0)),
            pl.BlockSpec((_P3_BT, D_HALF), lambda i: (i, 0)),
            pl.BlockSpec((1, 13), full),
            pl.BlockSpec((1, 13), full),
            pl.BlockSpec((13, D_HALF), full),
            pl.BlockSpec((1, D_HALF), full),
            pl.BlockSpec((1, D_HALF), full),
            pl.BlockSpec((D_MODEL, D_MODEL), full),
            pl.BlockSpec((1, D_MODEL), full),
        ],
        out_specs=pl.BlockSpec((_P3_BT, D_MODEL), lambda i: (i, 0)),
        out_shape=jax.ShapeDtypeStruct((T, D_MODEL), jnp.float32),
    )(x2, cat_sum, ln_gamma.reshape(1, 13), ln_beta.reshape(1, 13),
      W_cont, b_cont.reshape(1, D_HALF), b_cat.reshape(1, D_HALF),
      W_fin, b_fin.reshape(1, D_MODEL))


def kernel(x_cont, x_cat, ln_gamma, ln_beta, W_cont, b_cont, tables, W_cat,
           b_cat, W_fin, b_fin):
    w_cat3 = W_cat.reshape(N_CAT, EMB, D_HALF)
    proj = _project_tables(tables, w_cat3).reshape(N_CAT * VOCAB, D_HALF)

    idx_flat = (x_cat.reshape(T, N_CAT).astype(jnp.int32)
                + (jnp.arange(N_CAT, dtype=jnp.int32) * VOCAB)).reshape(-1)
    cat_sum = _gather_sum(idx_flat, proj)

    x2 = x_cont.reshape(T, 13)
    out = _finalize(x2, cat_sum, ln_gamma, ln_beta, W_cont, b_cont, b_cat,
                    W_fin, b_fin)
    return out.reshape(B, S, D_MODEL)


# speedup vs baseline: 7.2652x; 1.0275x over previous
import functools

import jax
import jax.numpy as jnp
from jax import lax
from jax.experimental import pallas as pl
from jax.experimental.pallas import tpu as pltpu
from jax.experimental.pallas import tpu_sc as plsc

N_CAT = 26
VOCAB = 100000
EMB = 50
D_HALF = 64
D_MODEL = 128
B, S = 4096, 50
T = B * S

_P1_BLK = 10000


def _proj_body(tlo_ref, thi_ref, w_ref, out_ref):
    dlo = jnp.dot(tlo_ref[0], w_ref[0], preferred_element_type=jnp.float32)
    dhi = jnp.dot(thi_ref[0], w_ref[0], preferred_element_type=jnp.float32)
    out_ref[...] = jnp.concatenate([dlo, dhi], axis=1)


def _project_tables(tables, w_cat3):
    v2 = VOCAB // 2
    nblk = v2 // _P1_BLK
    return pl.pallas_call(
        _proj_body,
        grid=(N_CAT, nblk),
        in_specs=[
            pl.BlockSpec((1, _P1_BLK, EMB), lambda i, j: (i, j, 0)),
            pl.BlockSpec((1, _P1_BLK, EMB),
                         lambda i, j, _n=nblk: (i, j + _n, 0)),
            pl.BlockSpec((1, EMB, D_HALF), lambda i, j: (i, 0, 0)),
        ],
        out_specs=pl.BlockSpec((_P1_BLK, 2 * D_HALF),
                               lambda i, j, _n=nblk: (i * _n + j, 0)),
        out_shape=jax.ShapeDtypeStruct((N_CAT * v2, 2 * D_HALF),
                                       jnp.float32),
    )(tables, tables, w_cat3)


_NC, _NS, _L = 2, 16, 16
_NW = _NC * _NS
_TPW = T // _NW
_CT = 64
_GPC = _CT * N_CAT
_NGB = _GPC // 128
_NCH = _TPW // _CT


def _gather_sum_body(idx_hbm, proj_hbm, out_hbm, idx_v, rows_v, acc_v, sem):
    wid = lax.axis_index("s") * _NC + lax.axis_index("c")
    base = wid * _TPW

    def chunk(ci, carry):
        tok0 = base + ci * _CT
        pltpu.sync_copy(idx_hbm.at[pl.ds(tok0 * N_CAT, _GPC)], idx_v)
        cps = [
            pltpu.async_copy(
                proj_hbm.at[idx_v.at[pl.ds(g * 128, 128)]],
                rows_v.at[pl.ds(g * 128, 128)],
                sem,
            )
            for g in range(_NGB)
        ]
        for cp in cps:
            cp.wait()

        def tok(t, c2):
            r0 = t * N_CAT
            for k in range(D_HALF // _L):
                s = rows_v[r0, pl.ds(k * _L, _L)]
                for j in range(1, N_CAT):
                    s = s + rows_v[r0 + j, pl.ds(k * _L, _L)]
                acc_v[t, pl.ds(k * _L, _L)] = s
            return c2

        lax.fori_loop(0, _CT, tok, 0, unroll=False)
        pltpu.sync_copy(acc_v, out_hbm.at[pl.ds(tok0, _CT)])
        return carry

    lax.fori_loop(0, _NCH, chunk, 0, unroll=False)


def _gather_sum(idx_flat, proj):
    mesh = plsc.VectorSubcoreMesh(core_axis_name="c", subcore_axis_name="s")
    return pl.kernel(
        _gather_sum_body,
        mesh=mesh,
        compiler_params=pltpu.CompilerParams(use_tc_tiling_on_sc=False),
        out_type=jax.ShapeDtypeStruct((T, D_HALF), jnp.float32),
        scratch_types=[
            pltpu.VMEM((_GPC,), jnp.int32),
            pltpu.VMEM((_GPC, D_HALF), jnp.float32),
            pltpu.VMEM((_CT, D_HALF), jnp.float32),
            pltpu.SemaphoreType.DMA,
        ],
    )(idx_flat, proj)


_P3_BT = 4096


def _final_body(x_ref, cs_ref, g_ref, b_ref, wc_ref, bc_ref, bcat_ref,
                wf_ref, bf_ref, out_ref):
    x = x_ref[...]
    mean = jnp.mean(x, axis=1, keepdims=True)
    cen = x - mean
    var = jnp.mean(cen * cen, axis=1, keepdims=True)
    xn = cen * lax.rsqrt(var + 1e-5) * g_ref[...] + b_ref[...]
    ce = jnp.dot(xn, wc_ref[...],
                 preferred_element_type=jnp.float32) + bc_ref[...]
    cat = cs_ref[...] + bcat_ref[...]
    comb = jnp.concatenate([ce, cat], axis=1)
    out_ref[...] = jnp.dot(comb, wf_ref[...],
                           preferred_element_type=jnp.float32) + bf_ref[...]


def _finalize(x2, cat_sum, ln_gamma, ln_beta, W_cont, b_cont, b_cat,
              W_fin, b_fin):
    nblk = T // _P3_BT
    full = lambda i: (0, 0)
    return pl.pallas_call(
        _final_body,
        grid=(nblk,),
        in_specs=[
            pl.BlockSpec((_P3_BT, 13), lambda i: (i, 0)),
            pl.BlockSpec((_P3_BT, D_HALF), lambda i: (i, 0)),
            pl.BlockSpec((1, 13), full),
            pl.BlockSpec((1, 13), full),
            pl.BlockSpec((13, D_HALF), full),
            pl.BlockSpec((1, D_HALF), full),
            pl.BlockSpec((1, D_HALF), full),
            pl.BlockSpec((D_MODEL, D_MODEL), full),
            pl.BlockSpec((1, D_MODEL), full),
        ],
        out_specs=pl.BlockSpec((_P3_BT, D_MODEL), lambda i: (i, 0)),
        out_shape=jax.ShapeDtypeStruct((T, D_MODEL), jnp.float32),
    )(x2, cat_sum, ln_gamma.reshape(1, 13), ln_beta.reshape(1, 13),
      W_cont, b_cont.reshape(1, D_HALF), b_cat.reshape(1, D_HALF),
      W_fin, b_fin.reshape(1, D_MODEL))


def kernel(x_cont, x_cat, ln_gamma, ln_beta, W_cont, b_cont, tables, W_cat,
           b_cat, W_fin, b_fin):
    w_cat3 = W_cat.reshape(N_CAT, EMB, D_HALF)
    proj = _project_tables(tables, w_cat3).reshape(N_CAT * VOCAB, D_HALF)

    v2 = VOCAB // 2
    xc = x_cat.reshape(T, N_CAT).astype(jnp.int32)
    packed = 2 * (xc % v2) + (xc // v2)
    idx_flat = (packed
                + (jnp.arange(N_CAT, dtype=jnp.int32) * VOCAB)).reshape(-1)
    cat_sum = _gather_sum(idx_flat, proj)

    x2 = x_cont.reshape(T, 13)
    out = _finalize(x2, cat_sum, ln_gamma, ln_beta, W_cont, b_cont, b_cat,
                    W_fin, b_fin)
    return out.reshape(B, S, D_MODEL)
